# Initial kernel scaffold; baseline (speedup 1.0000x reference)
import jax
import jax.numpy as jnp
from jax.experimental import pallas as pl


def kernel(target_wids, context_wids, emb):
    raise NotImplementedError("write your pallas kernel here")


# speedup vs baseline: 1.1076x; 1.1076x over previous
import functools

import jax
import jax.numpy as jnp
from jax import lax
from jax.experimental import pallas as pl
from jax.experimental.pallas import tpu as pltpu
from jax.experimental.pallas import tpu_sc as plsc

_VOCAB_SIZE = 1000000
_D = 64
_N_NEG = 10
_B = 16384
_L = 20

_NW = 32
_BPW = _B // _NW
_CB = 32
_NSUB = _BPW // _CB


def _sc_body(ctx_idx_hbm, neg_idx_hbm, tgt_idx_hbm, emb_hbm, out_hbm,
             ctx_idx_v, neg_idx_v, tgt_idx_v,
             ctx_rows, neg_rows, tgt_rows, pbuf, scoresW, sem):
  wid = lax.axis_index("s") * 2 + lax.axis_index("c")
  lane = lax.iota(jnp.int32, 16)
  zeros = jnp.zeros((16,), jnp.float32)

  def zrow(i, _):
    scoresW[11 + i // (_BPW // 16), pl.ds((i % (_BPW // 16)) * 16, 16)] = zeros
    return 0
  lax.fori_loop(0, 5 * (_BPW // 16), zrow, 0)

  lane16 = lane * 16

  def quad(q, _):
    off = wid * _BPW + q * 128
    pltpu.sync_copy(tgt_idx_hbm.at[pl.ds(off, 128)], tgt_idx_v)
    pltpu.async_copy(emb_hbm.at[tgt_idx_v.at[pl.ds(0, 128)]],
                     tgt_rows, sem).wait()

    for h in range(2):
      off = wid * (_BPW * _N_NEG) + (q * 2 + h) * (2 * _CB * _N_NEG)
      pltpu.sync_copy(neg_idx_hbm.at[pl.ds(off, 2 * _CB * _N_NEG)],
                      neg_idx_v)
      ncopies = [pltpu.async_copy(
          emb_hbm.at[neg_idx_v.at[pl.ds(c * 128, 128)]],
          neg_rows.at[pl.ds(c * 128, 128)], sem) for c in range(5)]
      for cp in ncopies:
        cp.wait()

      for s2 in range(2):
        s = (q * 2 + h) * 2 + s2
        off = wid * (_BPW * _L) + s * (_CB * _L)
        pltpu.sync_copy(ctx_idx_hbm.at[pl.ds(off, _CB * _L)], ctx_idx_v)
        ccopies = [pltpu.async_copy(
            emb_hbm.at[ctx_idx_v.at[pl.ds(c * 128, 128)]],
            ctx_rows.at[pl.ds(c * 128, 128)], sem) for c in range(5)]
        for cp in ccopies:
          cp.wait()

        tro = (h * 2 + s2) * _CB
        nro = s2 * _CB * _N_NEG

        def row_body(r, _):
          base = r * _L
          accs = [ctx_rows[base, pl.ds(c * 16, 16)] for c in range(4)]
          for l in range(1, _L):
            for c in range(4):
              accs[c] = accs[c] + ctx_rows[base + l, pl.ds(c * 16, 16)]
          p = tgt_rows[tro + r, pl.ds(0, 16)] * accs[0]
          for c in range(1, 4):
            p = p + tgt_rows[tro + r, pl.ds(c * 16, 16)] * accs[c]
          pbuf[pl.ds(r * 16, 16)] = p
          for j in range(_N_NEG):
            nrow = nro + r * _N_NEG + j
            p = neg_rows[nrow, pl.ds(0, 16)] * accs[0]
            for c in range(1, 4):
              p = p + neg_rows[nrow, pl.ds(c * 16, 16)] * accs[c]
            pbuf[pl.ds(((1 + j) * _CB + r) * 16, 16)] = p
          return 0
        lax.fori_loop(0, _CB, row_body, 0)

        for jp in range(1 + _N_NEG):
          for g in range(_CB // 16):
            pb = (jp * _CB + g * 16) * 16

            def red_step(k, acc):
              return acc + plsc.load_gather(pbuf, [lane16 + (pb + k)])

            acc = lax.fori_loop(0, 16, red_step, zeros)
            scoresW[jp, pl.ds(s * _CB + g * 16, 16)] = acc
    return 0

  lax.fori_loop(0, _NSUB // 4, quad, 0)
  pltpu.sync_copy(scoresW, out_hbm.at[:, pl.ds(wid * _BPW, _BPW)])


def _tc_body(scores_ref, out_ref):
  x = scores_ref[...]
  row = lax.broadcasted_iota(jnp.int32, x.shape, 0)
  sign = jnp.where(row == 0, 1.0, -1.0).astype(jnp.float32)
  v = sign * x * (1.0 / _L)
  z = jnp.minimum(v, 0.0) - jnp.log1p(jnp.exp(-jnp.abs(v)))
  z = jnp.where(row < 1 + _N_NEG, z, 0.0)
  out_ref[0, 0] = -jnp.sum(z)


def kernel(target_wids, context_wids, emb):
  batch_size = target_wids.shape[0]
  neg_wids = jax.random.randint(jax.random.key(1),
                                (batch_size, _N_NEG), 0, _VOCAB_SIZE - 1)

  ctx1d = context_wids.astype(jnp.int32).reshape(_B * _L)
  neg1d = neg_wids.astype(jnp.int32).reshape(_B * _N_NEG)
  tgt1d = target_wids.astype(jnp.int32).reshape(_B)

  mesh = plsc.VectorSubcoreMesh(core_axis_name="c", subcore_axis_name="s")
  sc = pl.kernel(
      _sc_body,
      out_type=jax.ShapeDtypeStruct((16, _B), jnp.float32),
      mesh=mesh,
      scratch_types=[
          pltpu.VMEM((_CB * _L,), jnp.int32),
          pltpu.VMEM((2 * _CB * _N_NEG,), jnp.int32),
          pltpu.VMEM((128,), jnp.int32),
          pltpu.VMEM((_CB * _L, _D), jnp.float32),
          pltpu.VMEM((2 * _CB * _N_NEG, _D), jnp.float32),
          pltpu.VMEM((128, _D), jnp.float32),
          pltpu.VMEM(((1 + _N_NEG) * _CB * 16,), jnp.float32),
          pltpu.VMEM((16, _BPW), jnp.float32),
          pltpu.SemaphoreType.DMA,
      ],
      compiler_params=pltpu.CompilerParams(needs_layout_passes=False,
                                           use_tc_tiling_on_sc=False),
  )
  scores = sc(ctx1d, neg1d, tgt1d, emb)

  loss = pl.pallas_call(
      _tc_body,
      out_shape=jax.ShapeDtypeStruct((1, 1), jnp.float32),
      out_specs=pl.BlockSpec(memory_space=pltpu.SMEM),
  )(scores)
  return loss[0, 0]


# speedup vs baseline: 1.1644x; 1.0512x over previous
import jax
import jax.numpy as jnp
from jax import lax
from jax.experimental import pallas as pl
from jax.experimental.pallas import tpu as pltpu
from jax.experimental.pallas import tpu_sc as plsc

_VOCAB_SIZE = 1000000
_D = 64
_N_NEG = 10
_B = 16384
_L = 20

_NW = 32
_BPW = _B // _NW
_CB = 32
_NSUB = _BPW // _CB
_NCTX = _CB * _L


def _sc_body(ctx_idx_hbm, neg_idx_hbm, tgt_idx_hbm, emb_hbm, out_hbm,
             ctx_idx_v, ngt_idx_v, ctx_rows, neg_rows, tgt_rows,
             csum, pbuf, scoresW, csem, nsem):
  wid = lax.axis_index("s") * 2 + lax.axis_index("c")
  lane = lax.iota(jnp.int32, 16)
  lane16 = lane * 16
  zeros = jnp.zeros((16,), jnp.float32)

  def zrow(i, _):
    scoresW[11 + i // (_BPW // 16), pl.ds((i % (_BPW // 16)) * 16, 16)] = zeros
    return 0
  lax.fori_loop(0, 5 * (_BPW // 16), zrow, 0)

  def fire_ctx(s):
    off = wid * (_BPW * _L) + s * _NCTX
    pltpu.sync_copy(ctx_idx_hbm.at[pl.ds(off, _NCTX)], ctx_idx_v)
    for c in range(5):
      pltpu.async_copy(emb_hbm.at[ctx_idx_v.at[pl.ds(c * 128, 128)]],
                       ctx_rows.at[pl.ds(c * 128, 128)], csem)

  def drain_ctx():
    for c in range(5):
      pltpu.make_async_copy(emb_hbm.at[ctx_idx_v.at[pl.ds(c * 128, 128)]],
                            ctx_rows.at[pl.ds(c * 128, 128)], csem).wait()

  fire_ctx(0)

  def subchunk(s, _):
    off = wid * (_BPW * _N_NEG) + s * (_CB * _N_NEG)
    pltpu.sync_copy(neg_idx_hbm.at[pl.ds(off, 384)],
                    ngt_idx_v.at[pl.ds(0, 384)])
    off = wid * _BPW + s * _CB
    pltpu.sync_copy(tgt_idx_hbm.at[pl.ds(off, 128)],
                    ngt_idx_v.at[pl.ds(384, 128)])
    drain_ctx()
    for c in range(3):
      pltpu.async_copy(emb_hbm.at[ngt_idx_v.at[pl.ds(c * 128, 128)]],
                       neg_rows.at[pl.ds(c * 128, 128)], nsem)
    pltpu.async_copy(emb_hbm.at[ngt_idx_v.at[pl.ds(384, 128)]],
                     tgt_rows, nsem)

    def row_sum(r, _):
      base = r * _L
      accs = [ctx_rows[base, pl.ds(c * 16, 16)] for c in range(4)]
      for l in range(1, _L):
        for c in range(4):
          accs[c] = accs[c] + ctx_rows[base + l, pl.ds(c * 16, 16)]
      for c in range(4):
        csum[r, pl.ds(c * 16, 16)] = accs[c]
      return 0
    lax.fori_loop(0, _CB, row_sum, 0)

    fire_ctx(jnp.minimum(s + 1, _NSUB - 1))

    for c in range(3):
      pltpu.make_async_copy(emb_hbm.at[ngt_idx_v.at[pl.ds(c * 128, 128)]],
                            neg_rows.at[pl.ds(c * 128, 128)], nsem).wait()
    pltpu.make_async_copy(emb_hbm.at[ngt_idx_v.at[pl.ds(384, 128)]],
                          tgt_rows, nsem).wait()

    def row_dot(r, _):
      cs = [csum[r, pl.ds(c * 16, 16)] for c in range(4)]
      p = tgt_rows[r, pl.ds(0, 16)] * cs[0]
      for c in range(1, 4):
        p = p + tgt_rows[r, pl.ds(c * 16, 16)] * cs[c]
      pbuf[pl.ds(r * 16, 16)] = p
      for j in range(_N_NEG):
        nrow = r * _N_NEG + j
        p = neg_rows[nrow, pl.ds(0, 16)] * cs[0]
        for c in range(1, 4):
          p = p + neg_rows[nrow, pl.ds(c * 16, 16)] * cs[c]
        pbuf[pl.ds(((1 + j) * _CB + r) * 16, 16)] = p
      return 0
    lax.fori_loop(0, _CB, row_dot, 0)

    for jp in range(1 + _N_NEG):
      for g in range(_CB // 16):
        pb = (jp * _CB + g * 16) * 16

        def red_step(k, acc):
          return acc + plsc.load_gather(pbuf, [lane16 + (pb + k)])

        acc = lax.fori_loop(0, 16, red_step, zeros)
        scoresW[jp, pl.ds(s * _CB + g * 16, 16)] = acc
    return 0

  lax.fori_loop(0, _NSUB, subchunk, 0)
  drain_ctx()
  pltpu.sync_copy(scoresW, out_hbm.at[:, pl.ds(wid * _BPW, _BPW)])


def _tc_body(scores_ref, out_ref):
  x = scores_ref[...]
  row = lax.broadcasted_iota(jnp.int32, x.shape, 0)
  sign = jnp.where(row == 0, 1.0, -1.0).astype(jnp.float32)
  v = sign * x * (1.0 / _L)
  z = jnp.minimum(v, 0.0) - jnp.log1p(jnp.exp(-jnp.abs(v)))
  z = jnp.where(row < 1 + _N_NEG, z, 0.0)
  out_ref[0, 0] = -jnp.sum(z)


def kernel(target_wids, context_wids, emb):
  batch_size = target_wids.shape[0]
  neg_wids = jax.random.randint(jax.random.key(1),
                                (batch_size, _N_NEG), 0, _VOCAB_SIZE - 1)

  ctx1d = context_wids.astype(jnp.int32).reshape(_B * _L)
  neg1d = jnp.concatenate(
      [neg_wids.astype(jnp.int32).reshape(_B * _N_NEG),
       jnp.zeros((64,), jnp.int32)])
  tgt1d = jnp.concatenate(
      [target_wids.astype(jnp.int32), jnp.zeros((96,), jnp.int32)])

  mesh = plsc.VectorSubcoreMesh(core_axis_name="c", subcore_axis_name="s")
  sc = pl.kernel(
      _sc_body,
      out_type=jax.ShapeDtypeStruct((16, _B), jnp.float32),
      mesh=mesh,
      scratch_types=[
          pltpu.VMEM((_NCTX,), jnp.int32),
          pltpu.VMEM((512,), jnp.int32),
          pltpu.VMEM((_NCTX, _D), jnp.float32),
          pltpu.VMEM((384, _D), jnp.float32),
          pltpu.VMEM((128, _D), jnp.float32),
          pltpu.VMEM((_CB, _D), jnp.float32),
          pltpu.VMEM(((1 + _N_NEG) * _CB * 16,), jnp.float32),
          pltpu.VMEM((16, _BPW), jnp.float32),
          pltpu.SemaphoreType.DMA,
          pltpu.SemaphoreType.DMA,
      ],
      compiler_params=pltpu.CompilerParams(needs_layout_passes=False,
                                           use_tc_tiling_on_sc=False),
  )
  scores = sc(ctx1d, neg1d, tgt1d, emb)

  loss = pl.pallas_call(
      _tc_body,
      out_shape=jax.ShapeDtypeStruct((1, 1), jnp.float32),
      out_specs=pl.BlockSpec(memory_space=pltpu.SMEM),
  )(scores)
  return loss[0, 0]


# speedup vs baseline: 1.1913x; 1.0231x over previous
import jax
import jax.numpy as jnp
from jax import lax
from jax.experimental import pallas as pl
from jax.experimental.pallas import tpu as pltpu
from jax.experimental.pallas import tpu_sc as plsc

_VOCAB_SIZE = 1000000
_D = 64
_N_NEG = 10
_B = 16384
_L = 20

_NW = 32
_BPW = _B // _NW
_CB = 32
_NSUB = _BPW // _CB
_NCTX = _CB * _L


def _sc_body(ctx_idx_hbm, neg_idx_hbm, tgt_idx_hbm, emb_hbm, out_hbm,
             ctx_idx_v, ngt_idx_v, ctx_rows, neg_rows, tgt_rows,
             csum, pbuf, scoresW, csem, nsem):
  wid = lax.axis_index("s") * 2 + lax.axis_index("c")
  lane = lax.iota(jnp.int32, 16)
  lane16 = lane * 16
  zeros = jnp.zeros((16,), jnp.float32)

  def zrow(i, _):
    scoresW[11 + i // (_BPW // 16), pl.ds((i % (_BPW // 16)) * 16, 16)] = zeros
    return 0
  lax.fori_loop(0, 5 * (_BPW // 16), zrow, 0)

  def fire_ctx(s):
    off = wid * (_BPW * _L) + s * _NCTX
    pltpu.sync_copy(ctx_idx_hbm.at[pl.ds(off, _NCTX)], ctx_idx_v)
    for c in range(5):
      pltpu.async_copy(emb_hbm.at[ctx_idx_v.at[pl.ds(c * 128, 128)]],
                       ctx_rows.at[pl.ds(c * 128, 128)], csem)

  def drain_ctx():
    for c in range(5):
      pltpu.make_async_copy(emb_hbm.at[ctx_idx_v.at[pl.ds(c * 128, 128)]],
                            ctx_rows.at[pl.ds(c * 128, 128)], csem).wait()

  fire_ctx(0)

  def subchunk(s, _):
    off = wid * (_BPW * _N_NEG) + s * (_CB * _N_NEG)
    pltpu.sync_copy(neg_idx_hbm.at[pl.ds(off, 384)],
                    ngt_idx_v.at[pl.ds(0, 384)])
    off = wid * _BPW + s * _CB
    pltpu.sync_copy(tgt_idx_hbm.at[pl.ds(off, 128)],
                    ngt_idx_v.at[pl.ds(384, 128)])
    drain_ctx()
    for c in range(3):
      pltpu.async_copy(emb_hbm.at[ngt_idx_v.at[pl.ds(c * 128, 128)]],
                       neg_rows.at[pl.ds(c * 128, 128)], nsem)
    pltpu.async_copy(emb_hbm.at[ngt_idx_v.at[pl.ds(384, 128)]],
                     tgt_rows, nsem)

    def row_sum(r2, _):
      for u in range(2):
        r = r2 * 2 + u
        base = r * _L
        a = [ctx_rows[base, pl.ds(c * 16, 16)] for c in range(4)]
        b = [ctx_rows[base + 1, pl.ds(c * 16, 16)] for c in range(4)]
        for l in range(2, _L, 2):
          for c in range(4):
            a[c] = a[c] + ctx_rows[base + l, pl.ds(c * 16, 16)]
            b[c] = b[c] + ctx_rows[base + l + 1, pl.ds(c * 16, 16)]
        for c in range(4):
          csum[r, pl.ds(c * 16, 16)] = a[c] + b[c]
      return 0
    lax.fori_loop(0, _CB // 2, row_sum, 0)

    fire_ctx(jnp.minimum(s + 1, _NSUB - 1))

    for c in range(3):
      pltpu.make_async_copy(emb_hbm.at[ngt_idx_v.at[pl.ds(c * 128, 128)]],
                            neg_rows.at[pl.ds(c * 128, 128)], nsem).wait()
    pltpu.make_async_copy(emb_hbm.at[ngt_idx_v.at[pl.ds(384, 128)]],
                          tgt_rows, nsem).wait()

    def row_dot(r, _):
      cs = [csum[r, pl.ds(c * 16, 16)] for c in range(4)]
      p = tgt_rows[r, pl.ds(0, 16)] * cs[0]
      for c in range(1, 4):
        p = p + tgt_rows[r, pl.ds(c * 16, 16)] * cs[c]
      pbuf[pl.ds(r * 16, 16)] = p
      for j in range(_N_NEG):
        nrow = r * _N_NEG + j
        p = neg_rows[nrow, pl.ds(0, 16)] * cs[0]
        for c in range(1, 4):
          p = p + neg_rows[nrow, pl.ds(c * 16, 16)] * cs[c]
        pbuf[pl.ds(((1 + j) * _CB + r) * 16, 16)] = p
      return 0
    lax.fori_loop(0, _CB, row_dot, 0)

    for jp in range(1 + _N_NEG):
      for g in range(_CB // 16):
        pb = (jp * _CB + g * 16) * 16

        def red_step(k2, carry):
          a0, a1 = carry
          k = k2 * 2
          a0 = a0 + plsc.load_gather(pbuf, [lane16 + (pb + k)])
          a1 = a1 + plsc.load_gather(pbuf, [lane16 + (pb + k + 1)])
          return (a0, a1)

        a0, a1 = lax.fori_loop(0, 8, red_step, (zeros, zeros))
        scoresW[jp, pl.ds(s * _CB + g * 16, 16)] = a0 + a1
    return 0

  lax.fori_loop(0, _NSUB, subchunk, 0)
  drain_ctx()
  pltpu.sync_copy(scoresW, out_hbm.at[:, pl.ds(wid * _BPW, _BPW)])


def _tc_body(scores_ref, out_ref):
  x = scores_ref[...]
  row = lax.broadcasted_iota(jnp.int32, x.shape, 0)
  sign = jnp.where(row == 0, 1.0, -1.0).astype(jnp.float32)
  v = sign * x * (1.0 / _L)
  z = jnp.minimum(v, 0.0) - jnp.log1p(jnp.exp(-jnp.abs(v)))
  z = jnp.where(row < 1 + _N_NEG, z, 0.0)
  out_ref[0, 0] = -jnp.sum(z)


def kernel(target_wids, context_wids, emb):
  batch_size = target_wids.shape[0]
  neg_wids = jax.random.randint(jax.random.key(1),
                                (batch_size, _N_NEG), 0, _VOCAB_SIZE - 1)

  ctx1d = context_wids.astype(jnp.int32).reshape(_B * _L)
  neg1d = jnp.concatenate(
      [neg_wids.astype(jnp.int32).reshape(_B * _N_NEG),
       jnp.zeros((64,), jnp.int32)])
  tgt1d = jnp.concatenate(
      [target_wids.astype(jnp.int32), jnp.zeros((96,), jnp.int32)])

  mesh = plsc.VectorSubcoreMesh(core_axis_name="c", subcore_axis_name="s")
  sc = pl.kernel(
      _sc_body,
      out_type=jax.ShapeDtypeStruct((16, _B), jnp.float32),
      mesh=mesh,
      scratch_types=[
          pltpu.VMEM((_NCTX,), jnp.int32),
          pltpu.VMEM((512,), jnp.int32),
          pltpu.VMEM((_NCTX, _D), jnp.float32),
          pltpu.VMEM((384, _D), jnp.float32),
          pltpu.VMEM((128, _D), jnp.float32),
          pltpu.VMEM((_CB, _D), jnp.float32),
          pltpu.VMEM(((1 + _N_NEG) * _CB * 16,), jnp.float32),
          pltpu.VMEM((16, _BPW), jnp.float32),
          pltpu.SemaphoreType.DMA,
          pltpu.SemaphoreType.DMA,
      ],
      compiler_params=pltpu.CompilerParams(needs_layout_passes=False,
                                           use_tc_tiling_on_sc=False),
  )
  scores = sc(ctx1d, neg1d, tgt1d, emb)

  loss = pl.pallas_call(
      _tc_body,
      out_shape=jax.ShapeDtypeStruct((1, 1), jnp.float32),
      out_specs=pl.BlockSpec(memory_space=pltpu.SMEM),
  )(scores)
  return loss[0, 0]


# speedup vs baseline: 1.1919x; 1.0005x over previous
import jax
import jax.numpy as jnp
from jax import lax
from jax.experimental import pallas as pl
from jax.experimental.pallas import tpu as pltpu
from jax.experimental.pallas import tpu_sc as plsc

_VOCAB_SIZE = 1000000
_D = 64
_N_NEG = 10
_B = 16384
_L = 20

_NW = 32
_BPW = _B // _NW
_CB = 32
_NSUB = _BPW // _CB
_NCTX = _CB * _L


def _sc_body(ctx_idx_hbm, neg_idx_hbm, tgt_idx_hbm, emb_hbm, out_hbm,
             ctx_idx_v, ngt_idx_v, ctx_rows, neg_rows, tgt_rows,
             csum, pbuf, scoresW, csem, nsem):
  wid = lax.axis_index("s") * 2 + lax.axis_index("c")
  lane = lax.iota(jnp.int32, 16)
  lane16 = lane * 16
  zeros = jnp.zeros((16,), jnp.float32)

  def zrow(i, _):
    scoresW[11 + i // (_BPW // 16), pl.ds((i % (_BPW // 16)) * 16, 16)] = zeros
    return 0
  lax.fori_loop(0, 5 * (_BPW // 16), zrow, 0)

  def fire_ctx(s):
    off = wid * (_BPW * _L) + s * _NCTX
    pltpu.sync_copy(ctx_idx_hbm.at[pl.ds(off, _NCTX)], ctx_idx_v)
    for c in range(5):
      pltpu.async_copy(emb_hbm.at[ctx_idx_v.at[pl.ds(c * 128, 128)]],
                       ctx_rows.at[pl.ds(c * 128, 128)], csem)

  def drain_ctx():
    for c in range(5):
      pltpu.make_async_copy(emb_hbm.at[ctx_idx_v.at[pl.ds(c * 128, 128)]],
                            ctx_rows.at[pl.ds(c * 128, 128)], csem).wait()

  fire_ctx(0)

  def subchunk(s, _):
    off = wid * (_BPW * _N_NEG) + s * (_CB * _N_NEG)
    pltpu.sync_copy(neg_idx_hbm.at[pl.ds(off, 384)],
                    ngt_idx_v.at[pl.ds(0, 384)])
    off = wid * _BPW + s * _CB
    pltpu.sync_copy(tgt_idx_hbm.at[pl.ds(off, 128)],
                    ngt_idx_v.at[pl.ds(384, 128)])
    drain_ctx()
    for c in range(3):
      pltpu.async_copy(emb_hbm.at[ngt_idx_v.at[pl.ds(c * 128, 128)]],
                       neg_rows.at[pl.ds(c * 128, 128)], nsem)
    pltpu.async_copy(emb_hbm.at[ngt_idx_v.at[pl.ds(384, 128)]],
                     tgt_rows, nsem)

    def row_sum(r2, _):
      for u in range(2):
        r = r2 * 2 + u
        base = r * _L
        a = [ctx_rows[base, pl.ds(c * 16, 16)] for c in range(4)]
        b = [ctx_rows[base + 1, pl.ds(c * 16, 16)] for c in range(4)]
        for l in range(2, _L, 2):
          for c in range(4):
            a[c] = a[c] + ctx_rows[base + l, pl.ds(c * 16, 16)]
            b[c] = b[c] + ctx_rows[base + l + 1, pl.ds(c * 16, 16)]
        for c in range(4):
          csum[r, pl.ds(c * 16, 16)] = a[c] + b[c]
      return 0
    lax.fori_loop(0, _CB // 2, row_sum, 0)

    fire_ctx(jnp.minimum(s + 1, _NSUB - 1))

    for c in range(3):
      pltpu.make_async_copy(emb_hbm.at[ngt_idx_v.at[pl.ds(c * 128, 128)]],
                            neg_rows.at[pl.ds(c * 128, 128)], nsem).wait()
    pltpu.make_async_copy(emb_hbm.at[ngt_idx_v.at[pl.ds(384, 128)]],
                          tgt_rows, nsem).wait()

    def row_dot(r2, _):
      for u in range(2):
        r = r2 * 2 + u
        cs = [csum[r, pl.ds(c * 16, 16)] for c in range(4)]
        p = tgt_rows[r, pl.ds(0, 16)] * cs[0]
        for c in range(1, 4):
          p = p + tgt_rows[r, pl.ds(c * 16, 16)] * cs[c]
        pbuf[pl.ds(r * 16, 16)] = p
        for j in range(_N_NEG):
          nrow = r * _N_NEG + j
          p = neg_rows[nrow, pl.ds(0, 16)] * cs[0]
          for c in range(1, 4):
            p = p + neg_rows[nrow, pl.ds(c * 16, 16)] * cs[c]
          pbuf[pl.ds(((1 + j) * _CB + r) * 16, 16)] = p
      return 0
    lax.fori_loop(0, _CB // 2, row_dot, 0)

    for jp in range(1 + _N_NEG):
      for g in range(_CB // 16):
        pb = (jp * _CB + g * 16) * 16

        def red_step(k2, carry):
          a0, a1 = carry
          k = k2 * 2
          a0 = a0 + plsc.load_gather(pbuf, [lane16 + (pb + k)])
          a1 = a1 + plsc.load_gather(pbuf, [lane16 + (pb + k + 1)])
          return (a0, a1)

        a0, a1 = lax.fori_loop(0, 8, red_step, (zeros, zeros))
        scoresW[jp, pl.ds(s * _CB + g * 16, 16)] = a0 + a1
    return 0

  lax.fori_loop(0, _NSUB, subchunk, 0)
  drain_ctx()
  pltpu.sync_copy(scoresW, out_hbm.at[:, pl.ds(wid * _BPW, _BPW)])


def _tc_body(scores_ref, out_ref):
  x = scores_ref[...]
  row = lax.broadcasted_iota(jnp.int32, x.shape, 0)
  sign = jnp.where(row == 0, 1.0, -1.0).astype(jnp.float32)
  v = sign * x * (1.0 / _L)
  z = jnp.minimum(v, 0.0) - jnp.log1p(jnp.exp(-jnp.abs(v)))
  z = jnp.where(row < 1 + _N_NEG, z, 0.0)
  out_ref[0, 0] = -jnp.sum(z)


def kernel(target_wids, context_wids, emb):
  batch_size = target_wids.shape[0]
  neg_wids = jax.random.randint(jax.random.key(1),
                                (batch_size, _N_NEG), 0, _VOCAB_SIZE - 1)

  ctx1d = context_wids.astype(jnp.int32).reshape(_B * _L)
  neg1d = jnp.concatenate(
      [neg_wids.astype(jnp.int32).reshape(_B * _N_NEG),
       jnp.zeros((64,), jnp.int32)])
  tgt1d = jnp.concatenate(
      [target_wids.astype(jnp.int32), jnp.zeros((96,), jnp.int32)])

  mesh = plsc.VectorSubcoreMesh(core_axis_name="c", subcore_axis_name="s")
  sc = pl.kernel(
      _sc_body,
      out_type=jax.ShapeDtypeStruct((16, _B), jnp.float32),
      mesh=mesh,
      scratch_types=[
          pltpu.VMEM((_NCTX,), jnp.int32),
          pltpu.VMEM((512,), jnp.int32),
          pltpu.VMEM((_NCTX, _D), jnp.float32),
          pltpu.VMEM((384, _D), jnp.float32),
          pltpu.VMEM((128, _D), jnp.float32),
          pltpu.VMEM((_CB, _D), jnp.float32),
          pltpu.VMEM(((1 + _N_NEG) * _CB * 16,), jnp.float32),
          pltpu.VMEM((16, _BPW), jnp.float32),
          pltpu.SemaphoreType.DMA,
          pltpu.SemaphoreType.DMA,
      ],
      compiler_params=pltpu.CompilerParams(needs_layout_passes=False,
                                           use_tc_tiling_on_sc=False),
  )
  scores = sc(ctx1d, neg1d, tgt1d, emb)

  loss = pl.pallas_call(
      _tc_body,
      out_shape=jax.ShapeDtypeStruct((1, 1), jnp.float32),
      out_specs=pl.BlockSpec(memory_space=pltpu.SMEM),
  )(scores)
  return loss[0, 0]
